# baseline (speedup 1.0000x reference)
import functools

import jax
import jax.numpy as jnp
from jax import lax
from jax.experimental import pallas as pl
from jax.experimental.pallas import tpu as pltpu
from jax.experimental.pallas import tpu_sc as plsc

GBUF = 3
OBUF = 2


@functools.lru_cache(maxsize=None)
def _make_gather(N: int, S: int, V: int, D: int):
    info = plsc.get_sparse_core_info()
    nc, ns = info.num_cores, info.num_subcores
    nw = nc * ns
    L = info.num_lanes
    W = 2 * D
    assert D % L == 0 and W == 128
    cols_per_w = N // nw
    nchunk = cols_per_w // W
    ntask = S * nchunk

    mesh = plsc.VectorSubcoreMesh(core_axis_name="c", subcore_axis_name="s")

    @functools.partial(
        pl.kernel,
        out_type=jax.ShapeDtypeStruct((S, D, N), jnp.float32),
        mesh=mesh,
        scratch_types=[
            pltpu.VMEM((S, cols_per_w), jnp.int32),
            pltpu.VMEM((GBUF, W), jnp.int32),
            pltpu.VMEM((GBUF, W), jnp.int32),
            pltpu.VMEM((GBUF, W, W), jnp.float32),
            pltpu.VMEM((OBUF, D, W + 1), jnp.float32),
            pltpu.SemaphoreType.DMA((GBUF,)),
            pltpu.SemaphoreType.DMA((OBUF,)),
        ],
        compiler_params=pltpu.CompilerParams(
            use_tc_tiling_on_sc=True, needs_layout_passes=False),
    )
    def gather_kernel(xt_hbm, tab_hbm, out_hbm, idx_v, pidx_v, par_v,
                      rows_v, out_v, gsem, osem):
        wid = lax.axis_index("s") * nc + lax.axis_index("c")
        col0 = wid * cols_per_w
        pltpu.sync_copy(xt_hbm.at[:, pl.ds(col0, cols_per_w)], idx_v)

        def prep(t, b):
            s = lax.div(t, nchunk)
            off = pl.multiple_of(lax.rem(t, nchunk) * W, W)
            for g in range(W // L):
                v = idx_v[s, pl.ds(off + L * g, L)]
                pidx_v[b, pl.ds(L * g, L)] = lax.shift_right_logical(v, 1)
                par_v[b, pl.ds(L * g, L)] = lax.shift_left(
                    lax.bitwise_and(v, 1), 6)

        def fire_gather(b):
            pltpu.async_copy(tab_hbm.at[pidx_v.at[b]], rows_v.at[b],
                             gsem.at[b])

        def wait_gather(b):
            pltpu.make_async_copy(tab_hbm.at[pidx_v.at[b]], rows_v.at[b],
                                  gsem.at[b]).wait()

        def out_slice(t):
            s = lax.div(t, nchunk)
            n0 = col0 + lax.rem(t, nchunk) * W
            return out_hbm.at[s, :, pl.ds(pl.multiple_of(n0, W), W)]

        def fire_out(t, bo):
            pltpu.async_copy(out_v.at[bo, :, pl.ds(0, W)], out_slice(t),
                             osem.at[bo])

        def wait_out(t, bo):
            pltpu.make_async_copy(out_v.at[bo, :, pl.ds(0, W)], out_slice(t),
                                  osem.at[bo]).wait()

        dvecs = [lax.iota(jnp.int32, L) + d0 for d0 in range(0, D, L)]

        def transpose_select(bg, bo):
            one = jnp.full((L,), 1, jnp.int32)
            j0 = jnp.full((L,), 0, jnp.int32)

            @pl.loop(0, W // 8, init_carry=j0)
            def _jgrp(jg, js):
                for _ in range(8):
                    parj = plsc.load_gather(par_v.at[bg], [js])
                    for dv in dvecs:
                        val = plsc.load_gather(rows_v.at[bg], [js, parj + dv])
                        plsc.store_scatter(out_v.at[bo], [dv, js], val)
                    js = js + one
                return js

        for t0 in range(GBUF - 1):
            prep(t0, t0)
            fire_gather(t0)

        @pl.loop(0, ntask)
        def _task(t):
            bg = lax.rem(t, GBUF)
            bo = lax.rem(t, OBUF)
            wait_gather(bg)

            @pl.when(t >= OBUF)
            def _():
                wait_out(t - OBUF, bo)

            transpose_select(bg, bo)
            fire_out(t, bo)
            tn = t + GBUF - 1

            @pl.when(tn < ntask)
            def _refill():
                bn = lax.rem(tn, GBUF)
                prep(tn, bn)
                fire_gather(bn)

        for k in range(OBUF):
            t = ntask - OBUF + k
            wait_out(t, t % OBUF)

    return gather_kernel


def kernel(x, table):
    n, s = x.shape
    V, D = table.shape
    xt = x.astype(jnp.int32).T
    tab2 = table.reshape(V // 2, 2 * D)
    out5 = _make_gather(n, s, V, D)(xt, tab2)
    return jnp.transpose(out5, (2, 0, 1))


# speedup vs baseline: 1.4516x; 1.4516x over previous
import functools

import jax
import jax.numpy as jnp
from jax import lax
from jax.experimental import pallas as pl
from jax.experimental.pallas import tpu as pltpu
from jax.experimental.pallas import tpu_sc as plsc

NBUF = 8
GDIST = 4


@functools.lru_cache(maxsize=None)
def _make_gather(N: int, S: int, V: int, D: int):
    info = plsc.get_sparse_core_info()
    nc, ns = info.num_cores, info.num_subcores
    nw = nc * ns
    assert N % (nw * NBUF) == 0
    rows_per_w = N // nw

    mesh = plsc.VectorSubcoreMesh(core_axis_name="c", subcore_axis_name="s")

    @functools.partial(
        pl.kernel,
        out_type=jax.ShapeDtypeStruct((N, S, D), jnp.float32),
        mesh=mesh,
        scratch_types=[
            pltpu.VMEM((rows_per_w, S), jnp.int32),
            pltpu.VMEM((NBUF, S, D), jnp.float32),
            pltpu.SemaphoreType.DMA((NBUF,)),
            pltpu.SemaphoreType.DMA((NBUF,)),
        ],
        compiler_params=pltpu.CompilerParams(use_tc_tiling_on_sc=False),
    )
    def gather_kernel(x_hbm, table_hbm, out_hbm, idx_v, rows_v, gsem, osem):
        wid = lax.axis_index("s") * nc + lax.axis_index("c")
        base = wid * rows_per_w
        pltpu.sync_copy(x_hbm.at[pl.ds(base, rows_per_w)], idx_v)

        def fire_gather(r, b):
            pltpu.async_copy(table_hbm.at[idx_v.at[r]], rows_v.at[b], gsem.at[b])

        def wait_gather(r, b):
            pltpu.make_async_copy(
                table_hbm.at[idx_v.at[r]], rows_v.at[b], gsem.at[b]
            ).wait()

        def fire_out(r, b):
            pltpu.async_copy(rows_v.at[b], out_hbm.at[base + r], osem.at[b])

        def wait_out(r, b):
            pltpu.make_async_copy(
                rows_v.at[b], out_hbm.at[base + r], osem.at[b]
            ).wait()

        for b in range(GDIST):
            fire_gather(b, b)

        @pl.loop(0, rows_per_w)
        def _step(r):
            b = lax.rem(r, NBUF)
            wait_gather(r, b)
            fire_out(r, b)
            rn = r + GDIST

            @pl.when(rn < rows_per_w)
            def _refill():
                bn = lax.rem(rn, NBUF)

                @pl.when(r >= NBUF - GDIST)
                def _():
                    wait_out(rn - NBUF, bn)

                fire_gather(rn, bn)

        for k in range(NBUF):
            r = rows_per_w - NBUF + k
            wait_out(r, r % NBUF)

    return gather_kernel


def kernel(x, table):
    n, s = x.shape
    V, D = table.shape
    return _make_gather(n, s, V, D)(x.astype(jnp.int32), table)


# speedup vs baseline: 1.4662x; 1.0100x over previous
import functools

import jax
import jax.numpy as jnp
from jax import lax
from jax.experimental import pallas as pl
from jax.experimental.pallas import tpu as pltpu
from jax.experimental.pallas import tpu_sc as plsc

D_MODEL = 64
CHUNK = 256
NBUF = 4
GDIST = 2


@functools.lru_cache(maxsize=None)
def _make_gather(B: int, V: int, D: int):
    info = plsc.get_sparse_core_info()
    nc, ns = info.num_cores, info.num_subcores
    nw = nc * ns
    assert B % (nw * CHUNK * NBUF) == 0
    b_per_w = B // nw
    n_chunks = b_per_w // CHUNK

    mesh = plsc.VectorSubcoreMesh(core_axis_name="c", subcore_axis_name="s")

    @functools.partial(
        pl.kernel,
        out_type=jax.ShapeDtypeStruct((B, D), jnp.float32),
        mesh=mesh,
        scratch_types=[
            pltpu.VMEM((n_chunks, CHUNK), jnp.int32),
            pltpu.VMEM((NBUF, CHUNK, D), jnp.float32),
            pltpu.SemaphoreType.DMA((NBUF,)),
            pltpu.SemaphoreType.DMA((NBUF,)),
        ],
        compiler_params=pltpu.CompilerParams(use_tc_tiling_on_sc=False),
    )
    def gather_kernel(x_hbm, table_hbm, out_hbm, idx_v, rows_v, gsem, osem):
        wid = lax.axis_index("s") * nc + lax.axis_index("c")
        base = wid * b_per_w
        pltpu.sync_copy(x_hbm.at[wid], idx_v)

        def fire_gather(c, b):
            pltpu.async_copy(table_hbm.at[idx_v.at[c]], rows_v.at[b], gsem.at[b])

        def wait_gather(c, b):
            pltpu.make_async_copy(
                table_hbm.at[idx_v.at[c]], rows_v.at[b], gsem.at[b]
            ).wait()

        def fire_out(c, b):
            pltpu.async_copy(
                rows_v.at[b], out_hbm.at[pl.ds(base + c * CHUNK, CHUNK)], osem.at[b]
            )

        def wait_out(c, b):
            pltpu.make_async_copy(
                rows_v.at[b], out_hbm.at[pl.ds(base + c * CHUNK, CHUNK)], osem.at[b]
            ).wait()

        for b in range(GDIST):
            fire_gather(b, b)

        @pl.loop(0, n_chunks)
        def _step(c):
            b = lax.rem(c, NBUF)
            wait_gather(c, b)
            fire_out(c, b)
            cn = c + GDIST

            @pl.when(cn < n_chunks)
            def _refill():
                bn = lax.rem(cn, NBUF)

                @pl.when(c >= NBUF - GDIST)
                def _():
                    wait_out(cn - NBUF, bn)

                fire_gather(cn, bn)

        for k in range(NBUF):
            c = n_chunks - NBUF + k
            wait_out(c, c % NBUF)

    return gather_kernel


def kernel(x, table):
    n, s = x.shape
    B = n * s
    V, D = table.shape
    info = plsc.get_sparse_core_info()
    nw = info.num_cores * info.num_subcores
    x_r = x.astype(jnp.int32).reshape(nw, B // (nw * CHUNK), CHUNK)
    out = _make_gather(B, V, D)(x_r, table)
    return out.reshape(n, s, D)
